# baseline (speedup 1.0000x reference)
import functools

import jax
import jax.numpy as jnp
from jax import lax
from jax.experimental import pallas as pl
from jax.experimental.pallas import tpu as pltpu
from jax.experimental.pallas import tpu_sc as plsc

_FIELD_DIM = 38462
_NUM_FIELDS = 26
_ROW = 32
_TOTAL = _FIELD_DIM * _NUM_FIELDS
_SUPER = _TOTAL // 4

_BATCH = 4096
_NW = 32
_BPW = _BATCH // _NW
_BLK = 16
_NBLKS = _BPW // _BLK


def _fm_body(xt_hbm, emb_hbm, out_hbm,
             idx_v, moff_v, buf0, buf1, rbuf, ybuf, sem0, sem1):
    c = lax.axis_index("c")
    s = lax.axis_index("s")
    w = s * 2 + c

    pltpu.sync_copy(xt_hbm.at[:, pl.ds(w * _BPW, _BPW)], idx_v)

    for j in range(_NUM_FIELDS):
        off = jnp.int32(j * _FIELD_DIM)
        for k in range(_BPW // 16):
            r = idx_v[j, pl.ds(k * 16, 16)] + off
            idx_v[j, pl.ds(k * 16, 16)] = lax.shift_right_logical(r, 2)
            moff_v[j, pl.ds(k * 16, 16)] = (r & 3) * _ROW

    bufs = (buf0, buf1)
    sems = (sem0, sem1)

    def start_block(b):
        bb = bufs[b % 2]
        sm = sems[b % 2]
        return [
            pltpu.async_copy(
                emb_hbm.at[idx_v.at[j, pl.ds(b * _BLK, _BLK)]], bb.at[j], sm)
            for j in range(_NUM_FIELDS)
        ]

    zeros = jnp.zeros((16,), jnp.float32)
    lanes = lax.iota(jnp.int32, 16)
    jsplats = [jnp.full((16,), j, jnp.int32) for j in range(_NUM_FIELDS)]
    pending = start_block(0)

    for b in range(_NBLKS):
        next_pending = start_block(b + 1) if b + 1 < _NBLKS else None
        for q in pending:
            q.wait()
        pending = next_pending
        bb = bufs[b % 2]

        def sbody(i, carry, bb=bb):
            isp = jnp.full((16,), 0, jnp.int32) + i
            bi = b * _BLK + i
            bisp = jnp.full((16,), 0, jnp.int32) + bi
            s1lo = zeros
            s2lo = zeros
            s1 = zeros
            s2 = zeros
            s3 = zeros
            for j in range(_NUM_FIELDS):
                msp = plsc.load_gather(moff_v, [jsplats[j], bisp])
                il = msp + lanes
                vlo = plsc.load_gather(bb, [jsplats[j], isp, il])
                vhi = plsc.load_gather(bb, [jsplats[j], isp, il + 16])
                s1lo = s1lo + vlo
                s2lo = s2lo + vlo * vlo
                q2 = vhi * vhi
                s1 = s1 + vhi
                s2 = s2 + q2
                s3 = s3 + q2 * vhi
            e2 = 0.5 * (s1lo * s1lo - s2lo)
            e3 = (s1 * s1 * s1 - 3.0 * s1 * s2 + 2.0 * s3) * (1.0 / 6.0)
            rbuf[pl.ds(i * 16, 16)] = e2 + e3
            return carry

        lax.fori_loop(0, _BLK, sbody, 0)

        acc = zeros
        for d in range(16):
            acc = acc + plsc.load_gather(rbuf, [lanes * 16 + jnp.int32(d)])
        ybuf[pl.ds(b * _BLK, 16)] = acc

    pltpu.sync_copy(ybuf, out_hbm.at[w])


def _lin_body(yfm_hbm, xt_hbm, lin_hbm, bias_hbm, out_hbm,
              idx_v, lin_v, ybuf, obuf, bias_v, sem):
    c = lax.axis_index("c")
    s = lax.axis_index("s")
    w = s * 2 + c

    pltpu.sync_copy(xt_hbm.at[:, pl.ds(w * _BPW, _BPW)], idx_v)
    pltpu.sync_copy(bias_hbm, bias_v)
    pltpu.sync_copy(yfm_hbm.at[w], ybuf)

    for j in range(_NUM_FIELDS):
        off = jnp.int32(j * _FIELD_DIM)
        for k in range(_BPW // 16):
            idx_v[j, pl.ds(k * 16, 16)] = idx_v[j, pl.ds(k * 16, 16)] + off

    descs = [
        pltpu.async_copy(lin_hbm.at[idx_v.at[j]], lin_v.at[j], sem)
        for j in range(_NUM_FIELDS)
    ]
    for q in descs:
        q.wait()

    bias16 = bias_v[...]
    for k in range(_BPW // 16):
        acc = ybuf[pl.ds(k * 16, 16)] + bias16
        for j in range(_NUM_FIELDS):
            acc = acc + lin_v[j, pl.ds(k * 16, 16)]
        obuf[pl.ds(k * 16, 16)] = 1.0 / (1.0 + jnp.exp(-acc))

    pltpu.sync_copy(obuf, out_hbm.at[pl.ds(w * _BPW, _BPW)])


@jax.jit
def _fm_sc(xt, emb4, lin1d, bias16):
    mesh = plsc.VectorSubcoreMesh(core_axis_name="c", subcore_axis_name="s")
    fa = functools.partial(
        pl.kernel,
        mesh=mesh,
        out_type=jax.ShapeDtypeStruct((_NW, _BPW), jnp.float32),
        scratch_types=[
            pltpu.VMEM((_NUM_FIELDS, _BPW), jnp.int32),
            pltpu.VMEM((_NUM_FIELDS, _BPW), jnp.int32),
            pltpu.VMEM((_NUM_FIELDS, _BLK, 128), jnp.float32),
            pltpu.VMEM((_NUM_FIELDS, _BLK, 128), jnp.float32),
            pltpu.VMEM((_BLK * 16,), jnp.float32),
            pltpu.VMEM((_BPW,), jnp.float32),
            pltpu.SemaphoreType.DMA,
            pltpu.SemaphoreType.DMA,
        ],
        compiler_params=pltpu.CompilerParams(
            needs_layout_passes=False, use_tc_tiling_on_sc=True),
    )(_fm_body)
    yfm = fa(xt, emb4)

    fb = functools.partial(
        pl.kernel,
        mesh=mesh,
        out_type=jax.ShapeDtypeStruct((_BATCH,), jnp.float32),
        scratch_types=[
            pltpu.VMEM((_NUM_FIELDS, _BPW), jnp.int32),
            pltpu.VMEM((_NUM_FIELDS, _BPW), jnp.float32),
            pltpu.VMEM((_BPW,), jnp.float32),
            pltpu.VMEM((_BPW,), jnp.float32),
            pltpu.VMEM((16,), jnp.float32),
            pltpu.SemaphoreType.DMA,
        ],
        compiler_params=pltpu.CompilerParams(
            needs_layout_passes=False, use_tc_tiling_on_sc=False),
    )(_lin_body)
    return fb(yfm, xt, lin1d, bias16)


def kernel(x, emb_table, lin_table, bias):
    xt = x.astype(jnp.int32).T
    emb4 = jnp.concatenate(
        [emb_table[m::4] for m in range(4)], axis=1)
    lin1d = lin_table.reshape(-1)
    bias16 = jnp.broadcast_to(bias.astype(jnp.float32), (16,))
    return _fm_sc(xt, emb4, lin1d, bias16)


# speedup vs baseline: 3.3794x; 3.3794x over previous
import functools

import jax
import jax.numpy as jnp
import numpy as np
from jax import lax
from jax.experimental import pallas as pl
from jax.experimental.pallas import tpu as pltpu
from jax.experimental.pallas import tpu_sc as plsc

_FIELD_DIM = 38462
_NUM_FIELDS = 26
_EMBED_DIM = 16
_TOTAL = _FIELD_DIM * _NUM_FIELDS

_BATCH = 4096
_NW = 32
_BPW = _BATCH // _NW

_PERM = np.stack([np.arange(16), np.arange(16) + 16], axis=1).reshape(-1)


def _fm_body(xt_hbm, emb_hbm, lin_hbm, bias_hbm, out_hbm,
             idx_v, lin_v, buf, rbuf, obuf, bias_v, sem_lin, sem):
    c = lax.axis_index("c")
    s = lax.axis_index("s")
    w = s * 2 + c

    pltpu.sync_copy(xt_hbm.at[:, pl.ds(w * _BPW, _BPW)], idx_v)
    pltpu.sync_copy(bias_hbm, bias_v)

    for j in range(_NUM_FIELDS):
        off = jnp.int32(j * _FIELD_DIM)
        for k in range(_BPW // 16):
            idx_v[j, pl.ds(k * 16, 16)] = idx_v[j, pl.ds(k * 16, 16)] + off

    emb_descs = [
        pltpu.async_copy(emb_hbm.at[idx_v.at[j]], buf.at[j], sem)
        for j in range(_NUM_FIELDS)
    ]
    lin_descs = [
        pltpu.async_copy(lin_hbm.at[idx_v.at[j]], lin_v.at[j], sem_lin)
        for j in range(_NUM_FIELDS)
    ]
    for q in emb_descs:
        q.wait()

    zeros = jnp.zeros((16,), jnp.float32)
    lanes = lax.iota(jnp.int32, 16)
    himask = jnp.full((16,), -65536, jnp.int32)

    def sbody(i, carry):
        s1lo = zeros
        s2lo = zeros
        s1 = zeros
        s2 = zeros
        s3 = zeros
        for j in range(_NUM_FIELDS):
            u = buf[j, i, pl.ds(0, 16)]
            vlo = plsc.bitcast(lax.shift_left(u, 16), jnp.float32)
            vhi = plsc.bitcast(u & himask, jnp.float32)
            s1lo = s1lo + vlo
            s2lo = s2lo + vlo * vlo
            q2 = vhi * vhi
            s1 = s1 + vhi
            s2 = s2 + q2
            s3 = s3 + q2 * vhi
        e2 = 0.5 * (s1lo * s1lo - s2lo)
        e3 = (s1 * s1 * s1 - 3.0 * s1 * s2 + 2.0 * s3) * (1.0 / 6.0)
        rbuf[pl.ds(i * 16, 16)] = e2 + e3
        return carry

    lax.fori_loop(0, _BPW, sbody, 0)

    for q in lin_descs:
        q.wait()

    for ch in range(_BPW // 16):
        acc = zeros
        for d in range(16):
            acc = acc + plsc.load_gather(
                rbuf, [lanes * 16 + jnp.int32(ch * 256 + d)])
        for j in range(_NUM_FIELDS):
            acc = acc + lin_v[j, pl.ds(ch * 16, 16)]
        y = acc + bias_v[...]
        obuf[pl.ds(ch * 16, 16)] = 1.0 / (1.0 + jnp.exp(-y))

    pltpu.sync_copy(obuf, out_hbm.at[pl.ds(w * _BPW, _BPW)])


@jax.jit
def _fm_sc(xt, embi, lin1d, bias16):
    mesh = plsc.VectorSubcoreMesh(core_axis_name="c", subcore_axis_name="s")
    f = functools.partial(
        pl.kernel,
        mesh=mesh,
        out_type=jax.ShapeDtypeStruct((_BATCH,), jnp.float32),
        scratch_types=[
            pltpu.VMEM((_NUM_FIELDS, _BPW), jnp.int32),
            pltpu.VMEM((_NUM_FIELDS, _BPW), jnp.float32),
            pltpu.VMEM((_NUM_FIELDS, _BPW, _EMBED_DIM), jnp.int32),
            pltpu.VMEM((_BPW * 16,), jnp.float32),
            pltpu.VMEM((_BPW,), jnp.float32),
            pltpu.VMEM((16,), jnp.float32),
            pltpu.SemaphoreType.DMA,
            pltpu.SemaphoreType.DMA,
        ],
        compiler_params=pltpu.CompilerParams(
            needs_layout_passes=False, use_tc_tiling_on_sc=False),
    )(_fm_body)
    return f(xt, embi, lin1d, bias16)


def kernel(x, emb_table, lin_table, bias):
    xt = x.astype(jnp.int32).T
    emb_bf = emb_table[:, _PERM].astype(jnp.bfloat16)
    embi = lax.bitcast_convert_type(
        emb_bf.reshape(_TOTAL, _EMBED_DIM, 2), jnp.int32)
    lin1d = lin_table.reshape(-1)
    bias16 = jnp.broadcast_to(bias.astype(jnp.float32), (16,))
    return _fm_sc(xt, embi, lin1d, bias16)


# speedup vs baseline: 7.0006x; 2.0716x over previous
import functools

import jax
import jax.numpy as jnp
from jax import lax
from jax.experimental import pallas as pl
from jax.experimental.pallas import tpu as pltpu
from jax.experimental.pallas import tpu_sc as plsc

_FIELD_DIM = 38462
_NUM_FIELDS = 26
_EMBED_DIM = 16
_TOTAL = _FIELD_DIM * _NUM_FIELDS

_BATCH = 4096
_NW = 32
_BPW = _BATCH // _NW


def _fm_body(xt_hbm, emb_hbm, lin_hbm, bias_hbm, out_hbm,
             idx_v, lin_v, buf, rbuf, obuf, bias_v, sem_lin, sem):
    c = lax.axis_index("c")
    s = lax.axis_index("s")
    w = s * 2 + c

    pltpu.sync_copy(xt_hbm.at[:, pl.ds(w * _BPW, _BPW)], idx_v)
    pltpu.sync_copy(bias_hbm, bias_v)

    for j in range(_NUM_FIELDS):
        off = jnp.int32(j * _FIELD_DIM)
        for k in range(_BPW // 16):
            idx_v[j, pl.ds(k * 16, 16)] = idx_v[j, pl.ds(k * 16, 16)] + off

    emb_descs = [
        pltpu.async_copy(emb_hbm.at[idx_v.at[j]], buf.at[j], sem)
        for j in range(_NUM_FIELDS)
    ]
    lin_descs = [
        pltpu.async_copy(lin_hbm.at[idx_v.at[j]], lin_v.at[j], sem_lin)
        for j in range(_NUM_FIELDS)
    ]
    for q in emb_descs:
        q.wait()

    zeros = jnp.zeros((16,), jnp.float32)
    lanes = lax.iota(jnp.int32, 16)
    himask = jnp.full((16,), -65536, jnp.int32)

    def sbody(i, carry):
        s1lo = zeros
        s2lo = zeros
        s1 = zeros
        s2 = zeros
        s3 = zeros
        for j in range(_NUM_FIELDS):
            u = buf[j, i, pl.ds(0, 16)]
            vlo = plsc.bitcast(lax.shift_left(u, 16), jnp.float32)
            vhi = plsc.bitcast(u & himask, jnp.float32)
            s1lo = s1lo + vlo
            s2lo = s2lo + vlo * vlo
            q2 = vhi * vhi
            s1 = s1 + vhi
            s2 = s2 + q2
            s3 = s3 + q2 * vhi
        e2 = 0.5 * (s1lo * s1lo - s2lo)
        e3 = (s1 * s1 * s1 - 3.0 * s1 * s2 + 2.0 * s3) * (1.0 / 6.0)
        rbuf[pl.ds(i * 16, 16)] = e2 + e3
        return carry

    lax.fori_loop(0, _BPW, sbody, 0)

    for q in lin_descs:
        q.wait()

    for ch in range(_BPW // 16):
        acc = zeros
        for d in range(16):
            acc = acc + plsc.load_gather(
                rbuf, [lanes * 16 + jnp.int32(ch * 256 + d)])
        for j in range(_NUM_FIELDS):
            acc = acc + lin_v[j, pl.ds(ch * 16, 16)]
        y = acc + bias_v[...]
        obuf[pl.ds(ch * 16, 16)] = 1.0 / (1.0 + jnp.exp(-y))

    pltpu.sync_copy(obuf, out_hbm.at[pl.ds(w * _BPW, _BPW)])


@jax.jit
def _fm_sc(xt, embi, lin1d, bias16):
    mesh = plsc.VectorSubcoreMesh(core_axis_name="c", subcore_axis_name="s")
    f = functools.partial(
        pl.kernel,
        mesh=mesh,
        out_type=jax.ShapeDtypeStruct((_BATCH,), jnp.float32),
        scratch_types=[
            pltpu.VMEM((_NUM_FIELDS, _BPW), jnp.int32),
            pltpu.VMEM((_NUM_FIELDS, _BPW), jnp.float32),
            pltpu.VMEM((_NUM_FIELDS, _BPW, _EMBED_DIM), jnp.int32),
            pltpu.VMEM((_BPW * 16,), jnp.float32),
            pltpu.VMEM((_BPW,), jnp.float32),
            pltpu.VMEM((16,), jnp.float32),
            pltpu.SemaphoreType.DMA,
            pltpu.SemaphoreType.DMA,
        ],
        compiler_params=pltpu.CompilerParams(
            needs_layout_passes=False, use_tc_tiling_on_sc=False),
    )(_fm_body)
    return f(xt, embi, lin1d, bias16)


def kernel(x, emb_table, lin_table, bias):
    xt = x.astype(jnp.int32).T
    emb_bf = emb_table.astype(jnp.bfloat16).reshape(_TOTAL, 2, _EMBED_DIM)
    emb_bf = emb_bf.transpose(0, 2, 1)
    embi = lax.bitcast_convert_type(emb_bf, jnp.int32)
    lin1d = lin_table.reshape(-1)
    bias16 = jnp.broadcast_to(bias.astype(jnp.float32), (16,))
    return _fm_sc(xt, embi, lin1d, bias16)


# speedup vs baseline: 8.6597x; 1.2370x over previous
import functools

import jax
import jax.numpy as jnp
from jax import lax
from jax.experimental import pallas as pl
from jax.experimental.pallas import tpu as pltpu
from jax.experimental.pallas import tpu_sc as plsc

_FIELD_DIM = 38462
_NUM_FIELDS = 26
_EMBED_DIM = 16
_TOTAL = _FIELD_DIM * _NUM_FIELDS

_BATCH = 4096
_NW = 32
_BPW = _BATCH // _NW


def _fm_body(xt_hbm, emb_hbm, lin_hbm, bias_hbm, out_hbm,
             idx_v, idxh_v, idxb_v, lin_v, buflo, bufhi, rbuf, obuf, bias_v,
             sem_lin, sem):
    c = lax.axis_index("c")
    s = lax.axis_index("s")
    w = s * 2 + c

    pltpu.sync_copy(xt_hbm.at[:, pl.ds(w * _BPW, _BPW)], idx_v)
    pltpu.sync_copy(bias_hbm, bias_v)

    for j in range(_NUM_FIELDS):
        off = jnp.int32(j * _FIELD_DIM)
        for k in range(_BPW // 16):
            sl = pl.ds(k * 16, 16)
            base = (idx_v[j, sl] + off)
            idxb_v[j, sl] = base
            idx_v[j, sl] = base * 2
            idxh_v[j, sl] = base * 2 + 1

    emb_descs = [
        pltpu.async_copy(emb_hbm.at[idx_v.at[j]], buflo.at[j], sem)
        for j in range(_NUM_FIELDS)
    ] + [
        pltpu.async_copy(emb_hbm.at[idxh_v.at[j]], bufhi.at[j], sem)
        for j in range(_NUM_FIELDS)
    ]
    lin_descs = [
        pltpu.async_copy(lin_hbm.at[idxb_v.at[j]], lin_v.at[j], sem_lin)
        for j in range(_NUM_FIELDS)
    ]
    for q in emb_descs:
        q.wait()

    zeros = jnp.zeros((16,), jnp.float32)
    lanes = lax.iota(jnp.int32, 16)

    def sbody(i, carry):
        s1lo = zeros
        s2lo = zeros
        s1 = zeros
        s2 = zeros
        s3 = zeros
        for j in range(_NUM_FIELDS):
            vlo = buflo[j, i, pl.ds(0, 16)]
            vhi = bufhi[j, i, pl.ds(0, 16)]
            s1lo = s1lo + vlo
            s2lo = s2lo + vlo * vlo
            q2 = vhi * vhi
            s1 = s1 + vhi
            s2 = s2 + q2
            s3 = s3 + q2 * vhi
        e2 = 0.5 * (s1lo * s1lo - s2lo)
        e3 = (s1 * s1 * s1 - 3.0 * s1 * s2 + 2.0 * s3) * (1.0 / 6.0)
        rbuf[pl.ds(i * 16, 16)] = e2 + e3
        return carry

    lax.fori_loop(0, _BPW, sbody, 0)

    for q in lin_descs:
        q.wait()

    for ch in range(_BPW // 16):
        acc = zeros
        for d in range(16):
            acc = acc + plsc.load_gather(
                rbuf, [lanes * 16 + jnp.int32(ch * 256 + d)])
        for j in range(_NUM_FIELDS):
            acc = acc + lin_v[j, pl.ds(ch * 16, 16)]
        y = acc + bias_v[...]
        obuf[pl.ds(ch * 16, 16)] = 1.0 / (1.0 + jnp.exp(-y))

    pltpu.sync_copy(obuf, out_hbm.at[pl.ds(w * _BPW, _BPW)])


@jax.jit
def _fm_sc(xt, embf, lin1d, bias16):
    mesh = plsc.VectorSubcoreMesh(core_axis_name="c", subcore_axis_name="s")
    f = functools.partial(
        pl.kernel,
        mesh=mesh,
        out_type=jax.ShapeDtypeStruct((_BATCH,), jnp.float32),
        scratch_types=[
            pltpu.VMEM((_NUM_FIELDS, _BPW), jnp.int32),
            pltpu.VMEM((_NUM_FIELDS, _BPW), jnp.int32),
            pltpu.VMEM((_NUM_FIELDS, _BPW), jnp.int32),
            pltpu.VMEM((_NUM_FIELDS, _BPW), jnp.float32),
            pltpu.VMEM((_NUM_FIELDS, _BPW, _EMBED_DIM), jnp.float32),
            pltpu.VMEM((_NUM_FIELDS, _BPW, _EMBED_DIM), jnp.float32),
            pltpu.VMEM((_BPW * 16,), jnp.float32),
            pltpu.VMEM((_BPW,), jnp.float32),
            pltpu.VMEM((16,), jnp.float32),
            pltpu.SemaphoreType.DMA,
            pltpu.SemaphoreType.DMA,
        ],
        compiler_params=pltpu.CompilerParams(
            needs_layout_passes=False, use_tc_tiling_on_sc=False),
    )(_fm_body)
    return f(xt, embf, lin1d, bias16)


def kernel(x, emb_table, lin_table, bias):
    xt = x.astype(jnp.int32).T
    embf = emb_table.reshape(_TOTAL * 2, _EMBED_DIM)
    lin1d = lin_table.reshape(-1)
    bias16 = jnp.broadcast_to(bias.astype(jnp.float32), (16,))
    return _fm_sc(xt, embf, lin1d, bias16)
